# baseline (speedup 1.0000x reference)
import functools

import jax
import jax.numpy as jnp
from jax import lax
from jax.experimental import pallas as pl
from jax.experimental.pallas import tpu as pltpu
from jax.experimental.pallas import tpu_sc as plsc

D = 768
NJ = D // 16
C = 64
NB = 4
S = 2048
F32 = jnp.float32


def _rsqrt16(x):
    i = plsc.bitcast(x, jnp.int32)
    i = jnp.int32(0x5F3759DF) - lax.shift_right_logical(i, 1)
    y = plsc.bitcast(i, F32)
    for _ in range(3):
        y = y * (1.5 - 0.5 * x * y * y)
    return y


def _sc_body(ids_hbm, sids_hbm, tok_hbm, pos_hbm, seg_hbm, gam_hbm, bet_hbm,
             out_hbm,
             tok_v, pos_v, seg_v, segd_v, gam_v, bet_v, ids_v, sids_v, sem):
    c = lax.axis_index("c")
    s = lax.axis_index("s")
    wid = s * 2 + c
    pbase = wid * C

    pltpu.sync_copy(seg_hbm, seg_v)
    pltpu.sync_copy(gam_hbm, gam_v)
    pltpu.sync_copy(bet_hbm, bet_v)
    pltpu.sync_copy(pos_hbm.at[pl.ds(pbase, C)], pos_v)
    for j in range(NJ):
        sl = pl.ds(j * 16, 16)
        segd_v[sl] = seg_v[1, sl] - seg_v[0, sl]

    for b in range(NB):
        tbase = b * S + pbase
        pltpu.sync_copy(ids_hbm.at[pl.ds(tbase, C)], ids_v)
        pltpu.sync_copy(sids_hbm.at[pl.ds(tbase, C)], sids_v.at[pl.ds(0, C)])
        pltpu.async_copy(tok_hbm.at[ids_v], tok_v, sem).wait()

        def token_body(t, carry):
            sv = sids_v[pl.ds(t, 16)]
            sidb = jnp.full((16,), sv[0], jnp.int32).astype(F32)
            acc_s = jnp.zeros((16,), F32)
            acc_q = jnp.zeros((16,), F32)
            for j in range(NJ):
                sl = pl.ds(j * 16, 16)
                v = tok_v[t, sl] + pos_v[t, sl] + (seg_v[0, sl] + sidb * segd_v[sl])
                tok_v[t, sl] = v
                acc_s = acc_s + v
                acc_q = acc_q + v * v
            mean = jnp.sum(acc_s) * (1.0 / D)
            var = jnp.sum(acc_q) * (1.0 / D) - mean * mean
            rs = _rsqrt16(jnp.full((16,), var + 1e-5, F32))
            mb = jnp.full((16,), mean, F32)
            for j in range(NJ):
                sl = pl.ds(j * 16, 16)
                v = tok_v[t, sl]
                tok_v[t, sl] = (v - mb) * rs * gam_v[sl] + bet_v[sl]
            return carry

        lax.fori_loop(0, C, token_body, 0)
        pltpu.sync_copy(tok_v, out_hbm.at[pl.ds(tbase, C)])


@functools.partial(jax.jit, static_argnames=())
def _sc_call(ids, sids, token_table, pos_table, seg_table, ln_gamma, ln_beta):
    mesh = plsc.VectorSubcoreMesh(core_axis_name="c", subcore_axis_name="s")
    run = functools.partial(
        pl.kernel,
        mesh=mesh,
        compiler_params=pltpu.CompilerParams(needs_layout_passes=False),
        out_type=jax.ShapeDtypeStruct((NB * S, D), F32),
        scratch_types=[
            pltpu.VMEM((C, D), F32),
            pltpu.VMEM((C, D), F32),
            pltpu.VMEM((2, D), F32),
            pltpu.VMEM((D,), F32),
            pltpu.VMEM((D,), F32),
            pltpu.VMEM((D,), F32),
            pltpu.VMEM((C,), jnp.int32),
            pltpu.VMEM((C + 16,), jnp.int32),
            pltpu.SemaphoreType.DMA,
        ],
    )(_sc_body)
    return run(ids, sids, token_table, pos_table, seg_table, ln_gamma, ln_beta)


def kernel(input_ids, segment_ids, token_table, pos_table, seg_table, ln_gamma, ln_beta):
    batch, seq = input_ids.shape
    ids = input_ids.reshape(-1)
    sids = segment_ids.reshape(-1)
    out = _sc_call(ids, sids, token_table, pos_table, seg_table, ln_gamma, ln_beta)
    return out.reshape(batch, seq, D)
